# baseline (speedup 1.0000x reference)
import functools

import jax
import jax.numpy as jnp
from jax import lax
from jax.experimental import pallas as pl
from jax.experimental.pallas import tpu as pltpu
from jax.experimental.pallas import tpu_sc as plsc

N = 10000
E = 320000
D = 128
H = 128

NC = 2
NS = 16
NW = NC * NS
EP = E // NW
CH = 125
NCH = EP // CH
NPAD = 10240
RPT = NPAD // NS

_MESH = plsc.VectorSubcoreMesh(core_axis_name="c", subcore_axis_name="s")


def _fill(ref, row, val):
    for i in range(8):
        if row is None:
            ref[pl.ds(i * 16, 16)] = jnp.full((16,), val, jnp.float32)
        else:
            ref[row, pl.ds(i * 16, 16)] = jnp.full((16,), val, jnp.float32)


_G = 8
_NG = NCH // _G


@functools.partial(
    pl.kernel,
    out_type=jax.ShapeDtypeStruct((NC, NPAD), jnp.float32),
    mesh=_MESH,
    scratch_types=[
        pltpu.VMEM((_NG, _G, CH), jnp.int32),
        pltpu.VMEM((128,), jnp.float32),
        pltpu.VMEM((RPT,), jnp.float32),
        pltpu.VMEM_SHARED((NPAD,), jnp.float32),
        pltpu.SemaphoreType.DMA,
    ],
)
def _deg_kernel(ei_hbm, deg_out, didx, ones_v, zbuf, deg_sh, sem):
    c = lax.axis_index("c")
    s = lax.axis_index("s")
    wid = s * NC + c
    _fill(ones_v, None, 1.0)
    for k in range(RPT // 16):
        zbuf[pl.ds(k * 16, 16)] = jnp.zeros((16,), jnp.float32)
    pltpu.sync_copy(zbuf, deg_sh.at[pl.ds(s * RPT, RPT)])
    pltpu.sync_copy(ei_hbm.at[1, wid], didx)
    plsc.subcore_barrier()

    def body(g, carry):
        for k in range(_G):
            pltpu.async_copy(ones_v.at[pl.ds(0, CH)],
                             deg_sh.at[didx.at[g, k]], sem, add=True)
        for k in range(_G):
            pltpu.make_async_copy(ones_v.at[pl.ds(0, CH)],
                                  deg_sh.at[didx.at[0, 0]], sem).wait()
        return carry

    lax.fori_loop(0, _NG, body, 0)
    plsc.subcore_barrier()
    pltpu.sync_copy(deg_sh.at[pl.ds(s * RPT, RPT)],
                    deg_out.at[c, pl.ds(s * RPT, RPT)])


@functools.partial(
    pl.kernel,
    out_type=jax.ShapeDtypeStruct((NC, NPAD, H), jnp.float32),
    mesh=_MESH,
    scratch_types=[
        pltpu.VMEM((2, 2, _G, CH), jnp.int32),
        pltpu.VMEM((CH, H), jnp.float32),
        pltpu.VMEM((CH, H), jnp.float32),
        pltpu.VMEM_SHARED((NPAD, H), jnp.float32),
        pltpu.SemaphoreType.DMA,
        pltpu.SemaphoreType.DMA,
        pltpu.SemaphoreType.DMA,
        pltpu.SemaphoreType.DMA,
        pltpu.SemaphoreType.DMA,
    ],
)
def _scatter_kernel(hs_hbm, ei_hbm, acc_out, idx, rows_a, rows_b,
                    acc_sh, sem_a, sem_b, sem_sa, sem_sb, sem_i):
    c = lax.axis_index("c")
    s = lax.axis_index("s")
    wid = s * NC + c

    def zrow(j, carry):
        for i in range(8):
            rows_a[j, pl.ds(i * 16, 16)] = jnp.zeros((16,), jnp.float32)
        return carry

    lax.fori_loop(0, CH, zrow, 0)
    for k in range(RPT // CH):
        pltpu.sync_copy(rows_a, acc_sh.at[pl.ds(s * RPT + k * CH, CH)])
    rem = RPT - (RPT // CH) * CH
    if rem:
        pltpu.sync_copy(rows_a.at[pl.ds(0, rem)],
                        acc_sh.at[pl.ds(s * RPT + RPT - rem, rem)])
    plsc.subcore_barrier()

    def issue_idx(g, slot, sem=None):
        if sem is None:
            pltpu.sync_copy(ei_hbm.at[0, wid, g], idx.at[slot, 0])
            pltpu.sync_copy(ei_hbm.at[1, wid, g], idx.at[slot, 1])
        else:
            pltpu.async_copy(ei_hbm.at[0, wid, g], idx.at[slot, 0], sem)
            pltpu.async_copy(ei_hbm.at[1, wid, g], idx.at[slot, 1], sem)

    def wait_idx():
        for _ in range(2):
            pltpu.make_async_copy(ei_hbm.at[0, wid, 0], idx.at[0, 0],
                                  sem_i).wait()

    def issue_gather(slot, k, buf, sem):
        pltpu.async_copy(hs_hbm.at[idx.at[slot, 0, k]], buf, sem)

    def wait_gather(buf, sem):
        pltpu.make_async_copy(hs_hbm.at[idx.at[0, 0, 0]], buf, sem).wait()

    def process_group(slot, tail):
        for k in range(0, _G, 2):
            issue_gather(slot, k + 1, rows_b, sem_b)
            wait_gather(rows_a, sem_a)
            pltpu.sync_copy(rows_a, acc_sh.at[idx.at[slot, 1, k]], add=True)
            if k + 2 < _G:
                issue_gather(slot, k + 2, rows_a, sem_a)
            else:
                tail()
            wait_gather(rows_b, sem_b)
            pltpu.sync_copy(rows_b, acc_sh.at[idx.at[slot, 1, k + 1]],
                            add=True)

    issue_idx(0, 0)
    issue_gather(0, 0, rows_a, sem_a)

    def gpair(u, carry):
        g0 = 2 * u
        issue_idx(g0 + 1, 1, sem_i)

        def tail0():
            wait_idx()
            issue_gather(1, 0, rows_a, sem_a)

        process_group(0, tail0)

        @pl.when(g0 + 2 < _NG)
        def _():
            issue_idx(g0 + 2, 0, sem_i)

        def tail1():
            @pl.when(g0 + 2 < _NG)
            def _():
                wait_idx()
                issue_gather(0, 0, rows_a, sem_a)

        process_group(1, tail1)
        return carry

    lax.fori_loop(0, _NG // 2, gpair, 0)
    plsc.subcore_barrier()
    pltpu.sync_copy(acc_sh.at[pl.ds(s * RPT, RPT)],
                    acc_out.at[c, pl.ds(s * RPT, RPT)])


_R = 1000


def _tc_mm_body(x_ref, w_ref, h_ref):
    h_ref[...] = jnp.dot(x_ref[...], w_ref[...],
                         preferred_element_type=jnp.float32)


def _tc_scale_body(h_ref, p0_ref, p1_ref, hs_ref, dinv_ref):
    deg = p0_ref[0] + p1_ref[0] + 1.0
    dinv = lax.rsqrt(deg)
    hs_ref[...] = h_ref[...] * dinv
    dinv_ref[...] = dinv


def _tc_mid_body(a0_ref, a1_ref, hs_ref, dinv_ref, b_ref, w_ref, out_ref):
    dinv = dinv_ref[...]
    z = (a0_ref[0] + a1_ref[0] + hs_ref[...]) * dinv + b_ref[...][None, :]
    x2 = jnp.maximum(z, 0.0)
    h2 = jnp.dot(x2, w_ref[...], preferred_element_type=jnp.float32)
    out_ref[...] = h2 * dinv


def _tc_fin_body(a0_ref, a1_ref, hs_ref, dinv_ref, b_ref, out_ref):
    z = ((a0_ref[0] + a1_ref[0] + hs_ref[...]) * dinv_ref[...]
         + b_ref[...][None, :])
    out_ref[...] = jnp.maximum(z, 0.0)


def _tc_mm(x, W1):
    return pl.pallas_call(
        _tc_mm_body,
        grid=(N // _R,),
        in_specs=[
            pl.BlockSpec((_R, D), lambda i: (i, 0)),
            pl.BlockSpec((D, H), lambda i: (0, 0)),
        ],
        out_specs=pl.BlockSpec((_R, H), lambda i: (i, 0)),
        out_shape=jax.ShapeDtypeStruct((N, H), jnp.float32),
    )(x, W1)


def _tc_scale(h1, degp3):
    return pl.pallas_call(
        _tc_scale_body,
        grid=(N // _R,),
        in_specs=[
            pl.BlockSpec((_R, H), lambda i: (i, 0)),
            pl.BlockSpec((1, _R, 1), lambda i: (0, i, 0)),
            pl.BlockSpec((1, _R, 1), lambda i: (1, i, 0)),
        ],
        out_specs=[
            pl.BlockSpec((_R, H), lambda i: (i, 0)),
            pl.BlockSpec((_R, 1), lambda i: (i, 0)),
        ],
        out_shape=[
            jax.ShapeDtypeStruct((N, H), jnp.float32),
            jax.ShapeDtypeStruct((N, 1), jnp.float32),
        ],
    )(h1, degp3, degp3)


def _tc_mid(acc, hs1, dinv, b1, W2):
    return pl.pallas_call(
        _tc_mid_body,
        grid=(N // _R,),
        in_specs=[
            pl.BlockSpec((1, _R, H), lambda i: (0, i, 0)),
            pl.BlockSpec((1, _R, H), lambda i: (1, i, 0)),
            pl.BlockSpec((_R, H), lambda i: (i, 0)),
            pl.BlockSpec((_R, 1), lambda i: (i, 0)),
            pl.BlockSpec((H,), lambda i: (0,)),
            pl.BlockSpec((H, H), lambda i: (0, 0)),
        ],
        out_specs=pl.BlockSpec((_R, H), lambda i: (i, 0)),
        out_shape=jax.ShapeDtypeStruct((N, H), jnp.float32),
    )(acc, acc, hs1, dinv, b1, W2)


def _tc_fin(acc, hs2, dinv, b2):
    return pl.pallas_call(
        _tc_fin_body,
        grid=(N // _R,),
        in_specs=[
            pl.BlockSpec((1, _R, H), lambda i: (0, i, 0)),
            pl.BlockSpec((1, _R, H), lambda i: (1, i, 0)),
            pl.BlockSpec((_R, H), lambda i: (i, 0)),
            pl.BlockSpec((_R, 1), lambda i: (i, 0)),
            pl.BlockSpec((H,), lambda i: (0,)),
        ],
        out_specs=pl.BlockSpec((_R, H), lambda i: (i, 0)),
        out_shape=jax.ShapeDtypeStruct((N, H), jnp.float32),
    )(acc, acc, hs2, dinv, b2)


def kernel(x, edge_index, W1, b1, W2, b2):
    ei = edge_index.reshape(2, NW, _NG, _G, CH)
    degp = _deg_kernel(ei)
    h1 = _tc_mm(x, W1)
    hs1, dinv = _tc_scale(h1, degp.reshape(2, NPAD, 1))
    acc1 = _scatter_kernel(hs1, ei)
    hs2 = _tc_mid(acc1, hs1, dinv, b1, W2)
    acc2 = _scatter_kernel(hs2, ei)
    return _tc_fin(acc2, hs2, dinv, b2)


# speedup vs baseline: 1.0104x; 1.0104x over previous
import functools

import jax
import jax.numpy as jnp
from jax import lax
from jax.experimental import pallas as pl
from jax.experimental.pallas import tpu as pltpu
from jax.experimental.pallas import tpu_sc as plsc

N = 10000
E = 320000
D = 128
H = 128

NC = 2
NS = 16
NW = NC * NS
EP = E // NW
CH = 125
NCH = EP // CH
NPAD = 10240
RPT = NPAD // NS

_MESH = plsc.VectorSubcoreMesh(core_axis_name="c", subcore_axis_name="s")


def _fill(ref, row, val):
    for i in range(8):
        if row is None:
            ref[pl.ds(i * 16, 16)] = jnp.full((16,), val, jnp.float32)
        else:
            ref[row, pl.ds(i * 16, 16)] = jnp.full((16,), val, jnp.float32)


_G = 8
_NG = NCH // _G


@functools.partial(
    pl.kernel,
    out_type=jax.ShapeDtypeStruct((NC, NPAD), jnp.float32),
    mesh=_MESH,
    scratch_types=[
        pltpu.VMEM((_NG, _G, CH), jnp.int32),
        pltpu.VMEM((128,), jnp.float32),
        pltpu.VMEM((RPT,), jnp.float32),
        pltpu.VMEM_SHARED((NPAD,), jnp.float32),
        pltpu.SemaphoreType.DMA,
    ],
)
def _deg_kernel(ei_hbm, deg_out, didx, ones_v, zbuf, deg_sh, sem):
    c = lax.axis_index("c")
    s = lax.axis_index("s")
    wid = s * NC + c
    _fill(ones_v, None, 1.0)
    for k in range(RPT // 16):
        zbuf[pl.ds(k * 16, 16)] = jnp.zeros((16,), jnp.float32)
    pltpu.sync_copy(zbuf, deg_sh.at[pl.ds(s * RPT, RPT)])
    pltpu.sync_copy(ei_hbm.at[1, wid], didx)
    plsc.subcore_barrier()

    def body(g, carry):
        for k in range(_G):
            pltpu.async_copy(ones_v.at[pl.ds(0, CH)],
                             deg_sh.at[didx.at[g, k]], sem, add=True)
        for k in range(_G):
            pltpu.make_async_copy(ones_v.at[pl.ds(0, CH)],
                                  deg_sh.at[didx.at[0, 0]], sem).wait()
        return carry

    lax.fori_loop(0, _NG, body, 0)
    plsc.subcore_barrier()
    pltpu.sync_copy(deg_sh.at[pl.ds(s * RPT, RPT)],
                    deg_out.at[c, pl.ds(s * RPT, RPT)])


@functools.partial(
    pl.kernel,
    out_type=jax.ShapeDtypeStruct((NC, NPAD, H), jnp.float32),
    mesh=_MESH,
    scratch_types=[
        pltpu.VMEM((2, 2, _G, CH), jnp.int32),
        pltpu.VMEM((CH, H), jnp.float32),
        pltpu.VMEM((CH, H), jnp.float32),
        pltpu.VMEM_SHARED((NPAD, H), jnp.float32),
        pltpu.SemaphoreType.DMA,
        pltpu.SemaphoreType.DMA,
        pltpu.SemaphoreType.DMA,
        pltpu.SemaphoreType.DMA,
        pltpu.SemaphoreType.DMA,
    ],
)
def _scatter_kernel(hs_hbm, ei_hbm, acc_out, idx, rows_a, rows_b,
                    acc_sh, sem_a, sem_b, sem_sa, sem_sb, sem_i):
    c = lax.axis_index("c")
    s = lax.axis_index("s")
    wid = s * NC + c

    def zrow(j, carry):
        for i in range(8):
            rows_a[j, pl.ds(i * 16, 16)] = jnp.zeros((16,), jnp.float32)
        return carry

    lax.fori_loop(0, CH, zrow, 0)
    for k in range(RPT // CH):
        pltpu.sync_copy(rows_a, acc_sh.at[pl.ds(s * RPT + k * CH, CH)])
    rem = RPT - (RPT // CH) * CH
    if rem:
        pltpu.sync_copy(rows_a.at[pl.ds(0, rem)],
                        acc_sh.at[pl.ds(s * RPT + RPT - rem, rem)])
    plsc.subcore_barrier()

    def issue_idx(g, slot, sem=None):
        if sem is None:
            pltpu.sync_copy(ei_hbm.at[0, wid, g], idx.at[slot, 0])
            pltpu.sync_copy(ei_hbm.at[1, wid, g], idx.at[slot, 1])
        else:
            pltpu.async_copy(ei_hbm.at[0, wid, g], idx.at[slot, 0], sem)
            pltpu.async_copy(ei_hbm.at[1, wid, g], idx.at[slot, 1], sem)

    def wait_idx():
        for _ in range(2):
            pltpu.make_async_copy(ei_hbm.at[0, wid, 0], idx.at[0, 0],
                                  sem_i).wait()

    def issue_gather(slot, k, buf, sem):
        pltpu.async_copy(hs_hbm.at[idx.at[slot, 0, k]], buf, sem)

    def wait_gather(buf, sem):
        pltpu.make_async_copy(hs_hbm.at[idx.at[0, 0, 0]], buf, sem).wait()

    def process_group(slot, tail):
        for k in range(0, _G, 2):
            issue_gather(slot, k + 1, rows_b, sem_b)
            wait_gather(rows_a, sem_a)
            pltpu.sync_copy(rows_a, acc_sh.at[idx.at[slot, 1, k]], add=True)
            if k + 2 < _G:
                issue_gather(slot, k + 2, rows_a, sem_a)
            else:
                tail()
            wait_gather(rows_b, sem_b)
            pltpu.sync_copy(rows_b, acc_sh.at[idx.at[slot, 1, k + 1]],
                            add=True)

    issue_idx(0, 0)
    issue_gather(0, 0, rows_a, sem_a)

    def gpair(u, carry):
        g0 = 2 * u
        issue_idx(g0 + 1, 1, sem_i)

        def tail0():
            wait_idx()
            issue_gather(1, 0, rows_a, sem_a)

        process_group(0, tail0)

        @pl.when(g0 + 2 < _NG)
        def _():
            issue_idx(g0 + 2, 0, sem_i)

        def tail1():
            @pl.when(g0 + 2 < _NG)
            def _():
                wait_idx()
                issue_gather(0, 0, rows_a, sem_a)

        process_group(1, tail1)
        return carry

    lax.fori_loop(0, _NG // 2, gpair, 0)
    plsc.subcore_barrier()
    pltpu.sync_copy(acc_sh.at[pl.ds(s * RPT, RPT)],
                    acc_out.at[c, pl.ds(s * RPT, RPT)])


_R = 1000


def _tc1_body(x_ref, w_ref, p0_ref, p1_ref, hs_ref, dinv_ref):
    deg = p0_ref[0] + p1_ref[0] + 1.0
    dinv = lax.rsqrt(deg)
    h = jnp.dot(x_ref[...], w_ref[...], preferred_element_type=jnp.float32)
    hs_ref[...] = h * dinv
    dinv_ref[...] = dinv


def _tc_mid_body(a0_ref, a1_ref, hs_ref, dinv_ref, b_ref, w_ref, out_ref):
    dinv = dinv_ref[...]
    z = (a0_ref[0] + a1_ref[0] + hs_ref[...]) * dinv + b_ref[...][None, :]
    x2 = jnp.maximum(z, 0.0)
    h2 = jnp.dot(x2, w_ref[...], preferred_element_type=jnp.float32)
    out_ref[...] = h2 * dinv


def _tc_fin_body(a0_ref, a1_ref, hs_ref, dinv_ref, b_ref, out_ref):
    z = ((a0_ref[0] + a1_ref[0] + hs_ref[...]) * dinv_ref[...]
         + b_ref[...][None, :])
    out_ref[...] = jnp.maximum(z, 0.0)


def _tc1(x, W1, degp3):
    return pl.pallas_call(
        _tc1_body,
        grid=(N // _R,),
        in_specs=[
            pl.BlockSpec((_R, D), lambda i: (i, 0)),
            pl.BlockSpec((D, H), lambda i: (0, 0)),
            pl.BlockSpec((1, _R, 1), lambda i: (0, i, 0)),
            pl.BlockSpec((1, _R, 1), lambda i: (1, i, 0)),
        ],
        out_specs=[
            pl.BlockSpec((_R, H), lambda i: (i, 0)),
            pl.BlockSpec((_R, 1), lambda i: (i, 0)),
        ],
        out_shape=[
            jax.ShapeDtypeStruct((N, H), jnp.float32),
            jax.ShapeDtypeStruct((N, 1), jnp.float32),
        ],
    )(x, W1, degp3, degp3)


def _tc_mid(acc, hs1, dinv, b1, W2):
    return pl.pallas_call(
        _tc_mid_body,
        grid=(N // _R,),
        in_specs=[
            pl.BlockSpec((1, _R, H), lambda i: (0, i, 0)),
            pl.BlockSpec((1, _R, H), lambda i: (1, i, 0)),
            pl.BlockSpec((_R, H), lambda i: (i, 0)),
            pl.BlockSpec((_R, 1), lambda i: (i, 0)),
            pl.BlockSpec((H,), lambda i: (0,)),
            pl.BlockSpec((H, H), lambda i: (0, 0)),
        ],
        out_specs=pl.BlockSpec((_R, H), lambda i: (i, 0)),
        out_shape=jax.ShapeDtypeStruct((N, H), jnp.float32),
    )(acc, acc, hs1, dinv, b1, W2)


def _tc_fin(acc, hs2, dinv, b2):
    return pl.pallas_call(
        _tc_fin_body,
        grid=(N // _R,),
        in_specs=[
            pl.BlockSpec((1, _R, H), lambda i: (0, i, 0)),
            pl.BlockSpec((1, _R, H), lambda i: (1, i, 0)),
            pl.BlockSpec((_R, H), lambda i: (i, 0)),
            pl.BlockSpec((_R, 1), lambda i: (i, 0)),
            pl.BlockSpec((H,), lambda i: (0,)),
        ],
        out_specs=pl.BlockSpec((_R, H), lambda i: (i, 0)),
        out_shape=jax.ShapeDtypeStruct((N, H), jnp.float32),
    )(acc, acc, hs2, dinv, b2)


def kernel(x, edge_index, W1, b1, W2, b2):
    ei = edge_index.reshape(2, NW, _NG, _G, CH)
    degp = _deg_kernel(ei)
    hs1, dinv = _tc1(x, W1, degp.reshape(2, NPAD, 1))
    acc1 = _scatter_kernel(hs1, ei)
    hs2 = _tc_mid(acc1, hs1, dinv, b1, W2)
    acc2 = _scatter_kernel(hs2, ei)
    return _tc_fin(acc2, hs2, dinv, b2)


# speedup vs baseline: 1.0170x; 1.0065x over previous
import functools

import jax
import jax.numpy as jnp
from jax import lax
from jax.experimental import pallas as pl
from jax.experimental.pallas import tpu as pltpu
from jax.experimental.pallas import tpu_sc as plsc

N = 10000
E = 320000
D = 128
H = 128

NC = 2
NS = 16
NW = NC * NS
EP = E // NW
CH = 125
NCH = EP // CH
NPAD = 10240
RPT = NPAD // NS

_MESH = plsc.VectorSubcoreMesh(core_axis_name="c", subcore_axis_name="s")


def _fill(ref, row, val):
    for i in range(8):
        if row is None:
            ref[pl.ds(i * 16, 16)] = jnp.full((16,), val, jnp.float32)
        else:
            ref[row, pl.ds(i * 16, 16)] = jnp.full((16,), val, jnp.float32)


_G = 8
_NG = NCH // _G


@functools.partial(
    pl.kernel,
    out_type=jax.ShapeDtypeStruct((NC, NPAD), jnp.float32),
    mesh=_MESH,
    scratch_types=[
        pltpu.VMEM((_NG, _G, CH), jnp.int32),
        pltpu.VMEM((128,), jnp.float32),
        pltpu.VMEM((RPT,), jnp.float32),
        pltpu.VMEM_SHARED((NPAD,), jnp.float32),
        pltpu.SemaphoreType.DMA,
    ],
)
def _deg_kernel(ei_hbm, deg_out, didx, ones_v, zbuf, deg_sh, sem):
    c = lax.axis_index("c")
    s = lax.axis_index("s")
    wid = s * NC + c
    _fill(ones_v, None, 1.0)
    for k in range(RPT // 16):
        zbuf[pl.ds(k * 16, 16)] = jnp.zeros((16,), jnp.float32)
    pltpu.sync_copy(zbuf, deg_sh.at[pl.ds(s * RPT, RPT)])
    pltpu.sync_copy(ei_hbm.at[1, wid], didx)
    plsc.subcore_barrier()

    def body(g, carry):
        for k in range(_G):
            pltpu.async_copy(ones_v.at[pl.ds(0, CH)],
                             deg_sh.at[didx.at[g, k]], sem, add=True)
        for k in range(_G):
            pltpu.make_async_copy(ones_v.at[pl.ds(0, CH)],
                                  deg_sh.at[didx.at[0, 0]], sem).wait()
        return carry

    lax.fori_loop(0, _NG, body, 0)
    plsc.subcore_barrier()
    pltpu.sync_copy(deg_sh.at[pl.ds(s * RPT, RPT)],
                    deg_out.at[c, pl.ds(s * RPT, RPT)])


@functools.partial(
    pl.kernel,
    out_type=jax.ShapeDtypeStruct((NC, NPAD, H), jnp.float32),
    mesh=_MESH,
    scratch_types=[
        pltpu.VMEM((2, 2, _G, CH), jnp.int32),
        pltpu.VMEM((CH, H), jnp.float32),
        pltpu.VMEM((CH, H), jnp.float32),
        pltpu.VMEM_SHARED((NPAD, H), jnp.float32),
        pltpu.SemaphoreType.DMA,
        pltpu.SemaphoreType.DMA,
        pltpu.SemaphoreType.DMA,
        pltpu.SemaphoreType.DMA,
        pltpu.SemaphoreType.DMA,
    ],
)
def _scatter_kernel(hs_hbm, ei_hbm, acc_out, idx, rows_a, rows_b,
                    acc_sh, sem_a, sem_b, sem_sa, sem_sb, sem_i):
    c = lax.axis_index("c")
    s = lax.axis_index("s")
    wid = s * NC + c

    def issue_idx(g, slot, sem=None):
        if sem is None:
            pltpu.sync_copy(ei_hbm.at[0, wid, g], idx.at[slot, 0])
            pltpu.sync_copy(ei_hbm.at[1, wid, g], idx.at[slot, 1])
        else:
            pltpu.async_copy(ei_hbm.at[0, wid, g], idx.at[slot, 0], sem)
            pltpu.async_copy(ei_hbm.at[1, wid, g], idx.at[slot, 1], sem)

    def wait_idx():
        for _ in range(2):
            pltpu.make_async_copy(ei_hbm.at[0, wid, 0], idx.at[0, 0],
                                  sem_i).wait()

    def issue_gather(slot, k, buf, sem):
        pltpu.async_copy(hs_hbm.at[idx.at[slot, 0, k]], buf, sem)

    def wait_gather(buf, sem):
        pltpu.make_async_copy(hs_hbm.at[idx.at[0, 0, 0]], buf, sem).wait()

    def process_group(slot, tail):
        for k in range(0, _G, 2):
            issue_gather(slot, k + 1, rows_b, sem_b)
            wait_gather(rows_a, sem_a)
            pltpu.sync_copy(rows_a, acc_sh.at[idx.at[slot, 1, k]], add=True)
            if k + 2 < _G:
                issue_gather(slot, k + 2, rows_a, sem_a)
            else:
                tail()
            wait_gather(rows_b, sem_b)
            pltpu.sync_copy(rows_b, acc_sh.at[idx.at[slot, 1, k + 1]],
                            add=True)

    issue_idx(0, 0, sem_i)

    def zrow(j, carry):
        for i in range(8):
            rows_b[j, pl.ds(i * 16, 16)] = jnp.zeros((16,), jnp.float32)
        return carry

    lax.fori_loop(0, CH, zrow, 0)
    for k in range(RPT // CH):
        pltpu.sync_copy(rows_b, acc_sh.at[pl.ds(s * RPT + k * CH, CH)])
    rem = RPT - (RPT // CH) * CH
    if rem:
        pltpu.sync_copy(rows_b.at[pl.ds(0, rem)],
                        acc_sh.at[pl.ds(s * RPT + RPT - rem, rem)])
    wait_idx()
    issue_gather(0, 0, rows_a, sem_a)
    plsc.subcore_barrier()

    def gpair(u, carry):
        g0 = 2 * u
        issue_idx(g0 + 1, 1, sem_i)

        def tail0():
            wait_idx()
            issue_gather(1, 0, rows_a, sem_a)

        process_group(0, tail0)

        @pl.when(g0 + 2 < _NG)
        def _():
            issue_idx(g0 + 2, 0, sem_i)

        def tail1():
            @pl.when(g0 + 2 < _NG)
            def _():
                wait_idx()
                issue_gather(0, 0, rows_a, sem_a)

        process_group(1, tail1)
        return carry

    lax.fori_loop(0, _NG // 2, gpair, 0)
    plsc.subcore_barrier()
    pltpu.sync_copy(acc_sh.at[pl.ds(s * RPT, RPT)],
                    acc_out.at[c, pl.ds(s * RPT, RPT)])


_R = 1000


def _tc1_body(x_ref, w_ref, p0_ref, p1_ref, hs_ref, dinv_ref):
    deg = p0_ref[0] + p1_ref[0] + 1.0
    dinv = lax.rsqrt(deg)
    h = jnp.dot(x_ref[...], w_ref[...], preferred_element_type=jnp.float32)
    hs_ref[...] = h * dinv
    dinv_ref[...] = dinv


def _tc_mid_body(a0_ref, a1_ref, hs_ref, dinv_ref, b_ref, w_ref, out_ref):
    dinv = dinv_ref[...]
    z = (a0_ref[0] + a1_ref[0] + hs_ref[...]) * dinv + b_ref[...][None, :]
    x2 = jnp.maximum(z, 0.0)
    h2 = jnp.dot(x2, w_ref[...], preferred_element_type=jnp.float32)
    out_ref[...] = h2 * dinv


def _tc_fin_body(a0_ref, a1_ref, hs_ref, dinv_ref, b_ref, out_ref):
    z = ((a0_ref[0] + a1_ref[0] + hs_ref[...]) * dinv_ref[...]
         + b_ref[...][None, :])
    out_ref[...] = jnp.maximum(z, 0.0)


def _tc1(x, W1, degp3):
    return pl.pallas_call(
        _tc1_body,
        grid=(N // _R,),
        in_specs=[
            pl.BlockSpec((_R, D), lambda i: (i, 0)),
            pl.BlockSpec((D, H), lambda i: (0, 0)),
            pl.BlockSpec((1, _R, 1), lambda i: (0, i, 0)),
            pl.BlockSpec((1, _R, 1), lambda i: (1, i, 0)),
        ],
        out_specs=[
            pl.BlockSpec((_R, H), lambda i: (i, 0)),
            pl.BlockSpec((_R, 1), lambda i: (i, 0)),
        ],
        out_shape=[
            jax.ShapeDtypeStruct((N, H), jnp.float32),
            jax.ShapeDtypeStruct((N, 1), jnp.float32),
        ],
    )(x, W1, degp3, degp3)


def _tc_mid(acc, hs1, dinv, b1, W2):
    return pl.pallas_call(
        _tc_mid_body,
        grid=(N // _R,),
        in_specs=[
            pl.BlockSpec((1, _R, H), lambda i: (0, i, 0)),
            pl.BlockSpec((1, _R, H), lambda i: (1, i, 0)),
            pl.BlockSpec((_R, H), lambda i: (i, 0)),
            pl.BlockSpec((_R, 1), lambda i: (i, 0)),
            pl.BlockSpec((H,), lambda i: (0,)),
            pl.BlockSpec((H, H), lambda i: (0, 0)),
        ],
        out_specs=pl.BlockSpec((_R, H), lambda i: (i, 0)),
        out_shape=jax.ShapeDtypeStruct((N, H), jnp.float32),
    )(acc, acc, hs1, dinv, b1, W2)


def _tc_fin(acc, hs2, dinv, b2):
    return pl.pallas_call(
        _tc_fin_body,
        grid=(N // _R,),
        in_specs=[
            pl.BlockSpec((1, _R, H), lambda i: (0, i, 0)),
            pl.BlockSpec((1, _R, H), lambda i: (1, i, 0)),
            pl.BlockSpec((_R, H), lambda i: (i, 0)),
            pl.BlockSpec((_R, 1), lambda i: (i, 0)),
            pl.BlockSpec((H,), lambda i: (0,)),
        ],
        out_specs=pl.BlockSpec((_R, H), lambda i: (i, 0)),
        out_shape=jax.ShapeDtypeStruct((N, H), jnp.float32),
    )(acc, acc, hs2, dinv, b2)


def kernel(x, edge_index, W1, b1, W2, b2):
    ei = edge_index.reshape(2, NW, _NG, _G, CH)
    degp = _deg_kernel(ei)
    hs1, dinv = _tc1(x, W1, degp.reshape(2, NPAD, 1))
    acc1 = _scatter_kernel(hs1, ei)
    hs2 = _tc_mid(acc1, hs1, dinv, b1, W2)
    acc2 = _scatter_kernel(hs2, ei)
    return _tc_fin(acc2, hs2, dinv, b2)
